# baseline (speedup 1.0000x reference)
import jax
import jax.numpy as jnp
from jax import lax
from jax.experimental import pallas as pl
from jax.experimental.pallas import tpu as pltpu
from jax.experimental.pallas import tpu_sc as plsc

BATCH = 16384
EMBED_DIM = 64
NUM_WORKERS = 32
B_PER_W = BATCH // NUM_WORKERS
CHUNK = 128
NCHUNK = B_PER_W // CHUNK
LANES = 16
GROUPS_PER_CHUNK = CHUNK // LANES


def _mf_body(uhalf_hbm, ihalf_hbm, uoff_hbm, ioff_hbm, table_hbm, out_hbm,
             uidx_v, iidx_v, off_v, rows_v, out_v, sem):
    cid = lax.axis_index("c")
    sid = lax.axis_index("s")
    wid = sid * 2 + cid
    base = wid * B_PER_W

    for j in range(NCHUNK):
        pltpu.sync_copy(uhalf_hbm.at[pl.ds(base + j * CHUNK, CHUNK)],
                        uidx_v.at[j])
        pltpu.sync_copy(ihalf_hbm.at[pl.ds(base + j * CHUNK, CHUNK)],
                        iidx_v.at[j])
    pltpu.sync_copy(uoff_hbm.at[pl.ds(base, B_PER_W)], off_v.at[0])
    pltpu.sync_copy(ioff_hbm.at[pl.ds(base, B_PER_W)], off_v.at[1])

    def fire(j, slot):
        u = pltpu.async_copy(table_hbm.at[uidx_v.at[j]],
                             rows_v.at[2 * slot], sem)
        i = pltpu.async_copy(table_hbm.at[iidx_v.at[j]],
                             rows_v.at[2 * slot + 1], sem)
        return (u, i)

    lane_iota = lax.iota(jnp.int32, LANES)
    pending = fire(0, 0)

    for j in range(NCHUNK):
        for c in pending:
            c.wait()
        if j + 1 < NCHUNK:
            nxt = fire(j + 1, (j + 1) % 2)
        ubuf = rows_v.at[2 * (j % 2)]
        ibuf = rows_v.at[2 * (j % 2) + 1]

        def group_body(g, _):
            rows = g * LANES + lane_iota
            ucol0 = plsc.load_gather(
                off_v, [jnp.zeros((LANES,), jnp.int32),
                        j * CHUNK + rows])
            icol0 = plsc.load_gather(
                off_v, [jnp.ones((LANES,), jnp.int32),
                        j * CHUNK + rows])
            acc0 = jnp.zeros((LANES,), jnp.float32)
            acc1 = jnp.zeros((LANES,), jnp.float32)
            for d in range(0, EMBED_DIM, 2):
                u0 = plsc.load_gather(ubuf, [rows, ucol0 + d])
                i0 = plsc.load_gather(ibuf, [rows, icol0 + d])
                u1 = plsc.load_gather(ubuf, [rows, ucol0 + (d + 1)])
                i1 = plsc.load_gather(ibuf, [rows, icol0 + (d + 1)])
                acc0 = acc0 + u0 * i0
                acc1 = acc1 + u1 * i1
            out_v[pl.ds(j * CHUNK + g * LANES, LANES)] = acc0 + acc1
            return 0

        lax.fori_loop(0, GROUPS_PER_CHUNK, group_body, 0)
        if j + 1 < NCHUNK:
            pending = nxt

    pltpu.sync_copy(out_v, out_hbm.at[pl.ds(base, B_PER_W)])


def _mf(uhalf, ihalf, uoff, ioff, table2):
    mesh = plsc.VectorSubcoreMesh(core_axis_name="c", subcore_axis_name="s")
    kern = pl.kernel(
        _mf_body,
        out_type=jax.ShapeDtypeStruct((BATCH,), jnp.float32),
        mesh=mesh,
        scratch_types=[
            pltpu.VMEM((NCHUNK, CHUNK), jnp.int32),
            pltpu.VMEM((NCHUNK, CHUNK), jnp.int32),
            pltpu.VMEM((2, B_PER_W), jnp.int32),
            pltpu.VMEM((4, CHUNK, 128), jnp.float32),
            pltpu.VMEM((B_PER_W,), jnp.float32),
            pltpu.SemaphoreType.DMA,
        ],
        compiler_params=pltpu.CompilerParams(needs_layout_passes=False,
                                             use_tc_tiling_on_sc=False),
    )
    return kern(uhalf, ihalf, uoff, ioff, table2)


@jax.jit
def _run(userID, ItemID, user_table):
    uid = userID.astype(jnp.int32)
    iid = ItemID.astype(jnp.int32)
    table2 = jnp.reshape(user_table, (user_table.shape[0] // 2, 128))
    return _mf(uid >> 1, iid >> 1,
               (uid & 1) << 6, (iid & 1) << 6, table2)


def kernel(userID, ItemID, user_table, item_table):
    del item_table
    return _run(userID, ItemID, user_table)


# speedup vs baseline: 1.4507x; 1.4507x over previous
import jax
import jax.numpy as jnp
from jax import lax
from jax.experimental import pallas as pl
from jax.experimental.pallas import tpu as pltpu
from jax.experimental.pallas import tpu_sc as plsc

BATCH = 16384
EMBED_DIM = 64
NUM_WORKERS = 32
B_PER_W = BATCH // NUM_WORKERS
CHUNK = 128
NCHUNK = B_PER_W // CHUNK
LANES = 16


def _mf_body(uhalf_hbm, ihalf_hbm, uoff_hbm, ioff_hbm, table_hbm, out_hbm,
             uidx_v, iidx_v, off_v, rows_v, part_v, out_v, sem):
    cid = lax.axis_index("c")
    sid = lax.axis_index("s")
    wid = sid * 2 + cid
    base = wid * B_PER_W

    for j in range(NCHUNK):
        pltpu.sync_copy(uhalf_hbm.at[pl.ds(base + j * CHUNK, CHUNK)],
                        uidx_v.at[j])
        pltpu.sync_copy(ihalf_hbm.at[pl.ds(base + j * CHUNK, CHUNK)],
                        iidx_v.at[j])
    pltpu.sync_copy(uoff_hbm.at[pl.ds(base, B_PER_W)], off_v.at[0])
    pltpu.sync_copy(ioff_hbm.at[pl.ds(base, B_PER_W)], off_v.at[1])

    def fire(j, slot):
        u = pltpu.async_copy(table_hbm.at[uidx_v.at[j]],
                             rows_v.at[2 * slot], sem)
        i = pltpu.async_copy(table_hbm.at[iidx_v.at[j]],
                             rows_v.at[2 * slot + 1], sem)
        return (u, i)

    lane_iota = lax.iota(jnp.int32, LANES)
    pending = fire(0, 0)

    for j in range(NCHUNK):
        for c in pending:
            c.wait()
        if j + 1 < NCHUNK:
            nxt = fire(j + 1, (j + 1) % 2)
        ubuf = rows_v.at[2 * (j % 2)]
        ibuf = rows_v.at[2 * (j % 2) + 1]

        def group_body(g, _):
            uoffs = off_v[0, pl.ds(j * CHUNK + g * LANES, LANES)]
            ioffs = off_v[1, pl.ds(j * CHUNK + g * LANES, LANES)]
            for m in range(LANES):
                b = g * LANES + m
                uo = uoffs[m]
                io = ioffs[m]
                acc = None
                for k in range(0, EMBED_DIM, LANES):
                    u = ubuf[b, pl.ds(uo + k, LANES)]
                    iv = ibuf[b, pl.ds(io + k, LANES)]
                    p = u * iv
                    acc = p if acc is None else acc + p
                part_v[m] = acc
            out_acc = None
            for k in range(LANES):
                col = plsc.load_gather(part_v, [lane_iota,
                                                jnp.full((LANES,), k,
                                                         jnp.int32)])
                out_acc = col if out_acc is None else out_acc + col
            out_v[pl.ds(j * CHUNK + g * LANES, LANES)] = out_acc
            return 0

        lax.fori_loop(0, CHUNK // LANES, group_body, 0)
        if j + 1 < NCHUNK:
            pending = nxt

    pltpu.sync_copy(out_v, out_hbm.at[pl.ds(base, B_PER_W)])


def _mf(uhalf, ihalf, uoff, ioff, table2):
    mesh = plsc.VectorSubcoreMesh(core_axis_name="c", subcore_axis_name="s")
    kern = pl.kernel(
        _mf_body,
        out_type=jax.ShapeDtypeStruct((BATCH,), jnp.float32),
        mesh=mesh,
        scratch_types=[
            pltpu.VMEM((NCHUNK, CHUNK), jnp.int32),
            pltpu.VMEM((NCHUNK, CHUNK), jnp.int32),
            pltpu.VMEM((2, B_PER_W), jnp.int32),
            pltpu.VMEM((4, CHUNK, 128), jnp.float32),
            pltpu.VMEM((LANES, LANES), jnp.float32),
            pltpu.VMEM((B_PER_W,), jnp.float32),
            pltpu.SemaphoreType.DMA,
        ],
        compiler_params=pltpu.CompilerParams(needs_layout_passes=False,
                                             use_tc_tiling_on_sc=True),
    )
    return kern(uhalf, ihalf, uoff, ioff, table2)


@jax.jit
def _run(userID, ItemID, user_table):
    uid = userID.astype(jnp.int32)
    iid = ItemID.astype(jnp.int32)
    table2 = jnp.reshape(user_table, (user_table.shape[0] // 2, 128))
    return _mf(uid >> 1, iid >> 1,
               (uid & 1) << 6, (iid & 1) << 6, table2)


def kernel(userID, ItemID, user_table, item_table):
    del item_table
    return _run(userID, ItemID, user_table)


# speedup vs baseline: 1.5100x; 1.0409x over previous
import jax
import jax.numpy as jnp
from jax import lax
from jax.experimental import pallas as pl
from jax.experimental.pallas import tpu as pltpu
from jax.experimental.pallas import tpu_sc as plsc

BATCH = 16384
EMBED_DIM = 64
NUM_WORKERS = 32
B_PER_W = BATCH // NUM_WORKERS
CHUNK = 128
NCHUNK = B_PER_W // CHUNK
LANES = 16
NSLOT = 3


def _mf_body(uhalf_hbm, ihalf_hbm, uoff_hbm, ioff_hbm, table_hbm, out_hbm,
             uidx_v, iidx_v, off_v, rows_v, part_v, out_v, isem, gsem):
    cid = lax.axis_index("c")
    sid = lax.axis_index("s")
    wid = sid * 2 + cid
    base = wid * B_PER_W

    idx_copies = []
    for j in range(NCHUNK):
        idx_copies.append((
            pltpu.async_copy(uhalf_hbm.at[pl.ds(base + j * CHUNK, CHUNK)],
                             uidx_v.at[j], isem),
            pltpu.async_copy(ihalf_hbm.at[pl.ds(base + j * CHUNK, CHUNK)],
                             iidx_v.at[j], isem),
        ))
    off_copies = (
        pltpu.async_copy(uoff_hbm.at[pl.ds(base, B_PER_W)], off_v.at[0],
                         isem),
        pltpu.async_copy(ioff_hbm.at[pl.ds(base, B_PER_W)], off_v.at[1],
                         isem),
    )

    def fire(j):
        slot = j % NSLOT
        u = pltpu.async_copy(table_hbm.at[uidx_v.at[j]],
                             rows_v.at[2 * slot], gsem)
        i = pltpu.async_copy(table_hbm.at[iidx_v.at[j]],
                             rows_v.at[2 * slot + 1], gsem)
        return (u, i)

    pending = []
    for j in range(NSLOT):
        for c in idx_copies[j]:
            c.wait()
        pending.append(fire(j))
    for c in off_copies:
        c.wait()

    lane_iota = lax.iota(jnp.int32, LANES)

    for j in range(NCHUNK):
        for c in pending[j]:
            c.wait()
        slot = j % NSLOT
        ubuf = rows_v.at[2 * slot]
        ibuf = rows_v.at[2 * slot + 1]

        def pair_body(h, _):
            for half in range(2):
                g = h * 2 + half
                uoffs = off_v[0, pl.ds(j * CHUNK + g * LANES, LANES)]
                ioffs = off_v[1, pl.ds(j * CHUNK + g * LANES, LANES)]
                for m in range(LANES):
                    b = g * LANES + m
                    uo = uoffs[m]
                    io = ioffs[m]
                    acc = None
                    for k in range(0, EMBED_DIM, LANES):
                        u = ubuf[b, pl.ds(uo + k, LANES)]
                        iv = ibuf[b, pl.ds(io + k, LANES)]
                        p = u * iv
                        acc = p if acc is None else acc + p
                    part_v[half * LANES + m] = acc
            for half in range(2):
                g = h * 2 + half
                out_acc = None
                for k in range(LANES):
                    col = plsc.load_gather(
                        part_v, [half * LANES + lane_iota,
                                 jnp.full((LANES,), k, jnp.int32)])
                    out_acc = col if out_acc is None else out_acc + col
                out_v[pl.ds(j * CHUNK + g * LANES, LANES)] = out_acc
            return 0

        lax.fori_loop(0, CHUNK // (2 * LANES), pair_body, 0)

        if j + NSLOT < NCHUNK:
            for c in idx_copies[j + NSLOT]:
                c.wait()
            pending.append(fire(j + NSLOT))

    pltpu.sync_copy(out_v, out_hbm.at[pl.ds(base, B_PER_W)])


def _mf(uhalf, ihalf, uoff, ioff, table2):
    mesh = plsc.VectorSubcoreMesh(core_axis_name="c", subcore_axis_name="s")
    kern = pl.kernel(
        _mf_body,
        out_type=jax.ShapeDtypeStruct((BATCH,), jnp.float32),
        mesh=mesh,
        scratch_types=[
            pltpu.VMEM((NCHUNK, CHUNK), jnp.int32),
            pltpu.VMEM((NCHUNK, CHUNK), jnp.int32),
            pltpu.VMEM((2, B_PER_W), jnp.int32),
            pltpu.VMEM((2 * NSLOT, CHUNK, 128), jnp.float32),
            pltpu.VMEM((2 * LANES, LANES), jnp.float32),
            pltpu.VMEM((B_PER_W,), jnp.float32),
            pltpu.SemaphoreType.DMA,
            pltpu.SemaphoreType.DMA,
        ],
        compiler_params=pltpu.CompilerParams(needs_layout_passes=False,
                                             use_tc_tiling_on_sc=True),
    )
    return kern(uhalf, ihalf, uoff, ioff, table2)


@jax.jit
def _run(userID, ItemID, user_table):
    uid = userID.astype(jnp.int32)
    iid = ItemID.astype(jnp.int32)
    table2 = jnp.reshape(user_table, (user_table.shape[0] // 2, 128))
    return _mf(uid >> 1, iid >> 1,
               (uid & 1) << 6, (iid & 1) << 6, table2)


def kernel(userID, ItemID, user_table, item_table):
    del item_table
    return _run(userID, ItemID, user_table)


# speedup vs baseline: 1.5228x; 1.0085x over previous
import jax
import jax.numpy as jnp
from jax import lax
from jax.experimental import pallas as pl
from jax.experimental.pallas import tpu as pltpu
from jax.experimental.pallas import tpu_sc as plsc

BATCH = 16384
EMBED_DIM = 64
NUM_WORKERS = 32
B_PER_W = BATCH // NUM_WORKERS
CHUNK = 128
NCHUNK = B_PER_W // CHUNK
LANES = 16
NSLOT = 3


def _mf_body(uid_hbm, iid_hbm, table_hbm, out_hbm,
             uraw_v, iraw_v, uidx_v, iidx_v, rows_v, part_v, out_v,
             isem, gsem):
    cid = lax.axis_index("c")
    sid = lax.axis_index("s")
    wid = sid * 2 + cid
    base = wid * B_PER_W

    raw_copies = (
        pltpu.async_copy(uid_hbm.at[pl.ds(base, B_PER_W)], uraw_v, isem),
        pltpu.async_copy(iid_hbm.at[pl.ds(base, B_PER_W)], iraw_v, isem),
    )

    def prep(j):
        for m in range(CHUNK // LANES):
            o = j * CHUNK + m * LANES
            uidx_v.at[j][pl.ds(m * LANES, LANES)] = (
                uraw_v[pl.ds(o, LANES)] >> 1)
            iidx_v.at[j][pl.ds(m * LANES, LANES)] = (
                iraw_v[pl.ds(o, LANES)] >> 1)

    def fire(j):
        slot = j % NSLOT
        u = pltpu.async_copy(table_hbm.at[uidx_v.at[j]],
                             rows_v.at[2 * slot], gsem)
        i = pltpu.async_copy(table_hbm.at[iidx_v.at[j]],
                             rows_v.at[2 * slot + 1], gsem)
        return (u, i)

    for c in raw_copies:
        c.wait()
    pending = []
    for j in range(NSLOT):
        prep(j)
        pending.append(fire(j))

    lane_iota = lax.iota(jnp.int32, LANES)

    for j in range(NCHUNK):
        for c in pending[j]:
            c.wait()
        slot = j % NSLOT
        ubuf = rows_v.at[2 * slot]
        ibuf = rows_v.at[2 * slot + 1]

        def pair_body(h, _):
            for half in range(2):
                g = h * 2 + half
                o = j * CHUNK + g * LANES
                uoffs = (uraw_v[pl.ds(o, LANES)] & 1) << 6
                ioffs = (iraw_v[pl.ds(o, LANES)] & 1) << 6
                for m in range(LANES):
                    b = g * LANES + m
                    uo = uoffs[m]
                    io = ioffs[m]
                    acc = None
                    for k in range(0, EMBED_DIM, LANES):
                        u = ubuf[b, pl.ds(uo + k, LANES)]
                        iv = ibuf[b, pl.ds(io + k, LANES)]
                        p = u * iv
                        acc = p if acc is None else acc + p
                    part_v[half * LANES + m] = acc
            for half in range(2):
                g = h * 2 + half
                out_acc = None
                for k in range(LANES):
                    col = plsc.load_gather(
                        part_v, [half * LANES + lane_iota,
                                 jnp.full((LANES,), k, jnp.int32)])
                    out_acc = col if out_acc is None else out_acc + col
                out_v[pl.ds(j * CHUNK + g * LANES, LANES)] = out_acc
            return 0

        lax.fori_loop(0, CHUNK // (2 * LANES), pair_body, 0)

        if j + NSLOT < NCHUNK:
            prep(j + NSLOT)
            pending.append(fire(j + NSLOT))

    pltpu.sync_copy(out_v, out_hbm.at[pl.ds(base, B_PER_W)])


def _mf(uid, iid, table2):
    mesh = plsc.VectorSubcoreMesh(core_axis_name="c", subcore_axis_name="s")
    kern = pl.kernel(
        _mf_body,
        out_type=jax.ShapeDtypeStruct((BATCH,), jnp.float32),
        mesh=mesh,
        scratch_types=[
            pltpu.VMEM((B_PER_W,), jnp.int32),
            pltpu.VMEM((B_PER_W,), jnp.int32),
            pltpu.VMEM((NCHUNK, CHUNK), jnp.int32),
            pltpu.VMEM((NCHUNK, CHUNK), jnp.int32),
            pltpu.VMEM((2 * NSLOT, CHUNK, 128), jnp.float32),
            pltpu.VMEM((2 * LANES, LANES), jnp.float32),
            pltpu.VMEM((B_PER_W,), jnp.float32),
            pltpu.SemaphoreType.DMA,
            pltpu.SemaphoreType.DMA,
        ],
        compiler_params=pltpu.CompilerParams(needs_layout_passes=False,
                                             use_tc_tiling_on_sc=True),
    )
    return kern(uid, iid, table2)


@jax.jit
def _run(userID, ItemID, user_table):
    table2 = jnp.reshape(user_table, (user_table.shape[0] // 2, 128))
    return _mf(userID.astype(jnp.int32), ItemID.astype(jnp.int32), table2)


def kernel(userID, ItemID, user_table, item_table):
    del item_table
    return _run(userID, ItemID, user_table)
